# speedup vs baseline: 1.0806x; 1.0806x over previous
import functools

import jax
import jax.numpy as jnp
from jax import lax
from jax.experimental import pallas as pl
from jax.experimental.pallas import tpu as pltpu
from jax.experimental.pallas import tpu_sc as plsc

VOCAB = 30522
HIDDEN = 768
MAX_POS = 512
BATCH = 128
SEQ = 512

NC = 2
NS = 16
NW = NC * NS
S_PER_W = SEQ // NW
CB = 32
NCHUNK = BATCH // CB
LANES = 16
NJ = HIDDEN // LANES


def _sc_embed(ids_t, token_table, pos_table):
    mesh = plsc.VectorSubcoreMesh(core_axis_name="c", subcore_axis_name="s")

    @functools.partial(
        pl.kernel,
        mesh=mesh,
        out_type=jax.ShapeDtypeStruct((BATCH, SEQ, HIDDEN), jnp.float32),
        scratch_types=[
            pltpu.VMEM((S_PER_W, BATCH), jnp.int32),
            pltpu.VMEM((S_PER_W, HIDDEN), jnp.float32),
            pltpu.VMEM((CB, HIDDEN), jnp.float32),
            pltpu.SemaphoreType.DMA,
        ],
    )
    def k(ids_hbm, tok_hbm, pos_hbm, out_hbm, idx_v, pos_v, buf_v, gsem):
        wid = lax.axis_index("s") * NC + lax.axis_index("c")
        s0 = wid * S_PER_W
        pltpu.sync_copy(ids_hbm.at[pl.ds(s0, S_PER_W), :], idx_v)
        pltpu.sync_copy(pos_hbm.at[pl.ds(s0, S_PER_W), :], pos_v)

        def per_s(s_local, carry):
            pos_regs = [pos_v[s_local, pl.ds(j * LANES, LANES)]
                        for j in range(NJ)]

            def per_chunk(c, carry2):
                b0 = c * CB
                pltpu.async_copy(
                    tok_hbm.at[idx_v.at[s_local, pl.ds(b0, CB)]],
                    buf_v, gsem).wait()

                def add_row(vb, carry3):
                    for j in range(NJ):
                        sl = pl.ds(j * LANES, LANES)
                        buf_v[vb, sl] = buf_v[vb, sl] + pos_regs[j]
                    return carry3

                lax.fori_loop(0, CB, add_row, 0)
                pltpu.sync_copy(
                    buf_v, out_hbm.at[pl.ds(b0, CB), s0 + s_local, :])
                return carry2

            lax.fori_loop(0, NCHUNK, per_chunk, 0)
            return carry

        lax.fori_loop(0, S_PER_W, per_s, 0)

    return k(ids_t, token_table, pos_table)


def kernel(input_ids, token_table, pos_table):
    ids_t = input_ids.astype(jnp.int32).T
    return _sc_embed(ids_t, token_table, pos_table)


# speedup vs baseline: 1.7098x; 1.5823x over previous
import functools

import jax
import jax.numpy as jnp
from jax import lax
from jax.experimental import pallas as pl
from jax.experimental.pallas import tpu as pltpu
from jax.experimental.pallas import tpu_sc as plsc

VOCAB = 30522
HIDDEN = 768
MAX_POS = 512
BATCH = 128
SEQ = 512

NC = 2
NS = 16
NW = NC * NS
S_PER_W = SEQ // NW
CB = 32
NCHUNK = BATCH // CB
NBUF = 4
LANES = 16
NJ = HIDDEN // LANES


def _sc_embed(ids_t, token_table, pos_table):
    mesh = plsc.VectorSubcoreMesh(core_axis_name="c", subcore_axis_name="s")

    @functools.partial(
        pl.kernel,
        mesh=mesh,
        out_type=jax.ShapeDtypeStruct((BATCH, SEQ, HIDDEN), jnp.float32),
        scratch_types=[
            pltpu.VMEM((S_PER_W, BATCH), jnp.int32),
            pltpu.VMEM((S_PER_W, HIDDEN), jnp.float32),
            pltpu.VMEM((NBUF, CB, HIDDEN), jnp.float32),
        ] + [pltpu.SemaphoreType.DMA] * (2 * NBUF),
    )
    def k(ids_hbm, tok_hbm, pos_hbm, out_hbm, idx_v, pos_v, buf_v, *sems):
        gsems, osems = sems[:NBUF], sems[NBUF:]
        wid = lax.axis_index("s") * NC + lax.axis_index("c")
        s0 = wid * S_PER_W
        pltpu.sync_copy(ids_hbm.at[pl.ds(s0, S_PER_W), :], idx_v)
        pltpu.sync_copy(pos_hbm.at[pl.ds(s0, S_PER_W), :], pos_v)

        def g_start(sl, c, bslot):
            return pltpu.async_copy(
                tok_hbm.at[idx_v.at[sl, pl.ds(c * CB, CB)]],
                buf_v.at[bslot], gsems[bslot])

        def g_wait(sl, c, bslot):
            pltpu.make_async_copy(
                tok_hbm.at[idx_v.at[sl, pl.ds(c * CB, CB)]],
                buf_v.at[bslot], gsems[bslot]).wait()

        def s_start(sl, c, bslot):
            return pltpu.async_copy(
                buf_v.at[bslot],
                out_hbm.at[pl.ds(c * CB, CB), s0 + sl, :], osems[bslot])

        def s_wait(sl, c, bslot):
            pltpu.make_async_copy(
                buf_v.at[bslot],
                out_hbm.at[pl.ds(c * CB, CB), s0 + sl, :],
                osems[bslot]).wait()

        g_start(0, 0, 0)
        g_start(0, 1, 1)

        def per_k(sk, carry):
            for b in range(NBUF):
                g_wait(sk, b, b)
                if b < 2:
                    @pl.when(sk >= 1)
                    def _():
                        s_wait(sk - 1, b + 2, b + 2)
                    g_start(sk, b + 2, b + 2)
                else:
                    @pl.when(sk < S_PER_W - 1)
                    def _():
                        s_wait(sk, b - 2, b - 2)
                        g_start(sk + 1, b - 2, b - 2)
                pos_regs = [pos_v[sk, pl.ds(j * LANES, LANES)]
                            for j in range(NJ)]

                def add_row(vb, c3):
                    for j in range(NJ):
                        sl = pl.ds(j * LANES, LANES)
                        buf_v[b, vb, sl] = buf_v[b, vb, sl] + pos_regs[j]
                    return c3

                lax.fori_loop(0, CB, add_row, 0)
                s_start(sk, b, b)
            return carry

        lax.fori_loop(0, S_PER_W, per_k, 0)
        for b in range(NBUF):
            s_wait(S_PER_W - 1, b, b)

    return k(ids_t, token_table, pos_table)


def kernel(input_ids, token_table, pos_table):
    ids_t = input_ids.astype(jnp.int32).T
    return _sc_embed(ids_t, token_table, pos_table)
